# Initial kernel scaffold; baseline (speedup 1.0000x reference)
import jax
import jax.numpy as jnp
from jax.experimental import pallas as pl


def kernel(input, target):
    raise NotImplementedError("write your pallas kernel here")


# speedup vs baseline: 1.7951x; 1.7951x over previous
import functools
from math import log

import jax
import jax.numpy as jnp
from jax import lax
from jax.experimental import pallas as pl
from jax.experimental.pallas import tpu as pltpu

_IGNORE_INDEX = -100
_THRESH = -log(0.7)

_N_ROWS = 8192
_N_COLS = 4096
_BLOCK_ROWS = 256
_N_BLOCKS = _N_ROWS // _BLOCK_ROWS
_TOPN = int(_N_ROWS * 0.01)


def _ohem_body(x_ref, tgt_ref, out_ref, loss_ref):
    i = pl.program_id(0)

    x = x_ref[...]
    t = tgt_ref[0, 0, :]

    m = jnp.max(x, axis=1, keepdims=True)
    s = jnp.sum(jnp.exp(x - m), axis=1)
    lse = m[:, 0] + jnp.log(s)

    t_safe = jnp.clip(t, 0, _N_COLS - 1)
    col = lax.broadcasted_iota(jnp.int32, (_BLOCK_ROWS, _N_COLS), 1)
    picked = jnp.sum(jnp.where(col == t_safe[:, None], x, 0.0), axis=1)

    valid = t != _IGNORE_INDEX
    loss = jnp.where(valid, lse - picked, 0.0)
    loss_ref[pl.ds(i, 1), :] = loss[None, :]

    @pl.when(i == _N_BLOCKS - 1)
    def _():
        all_loss = loss_ref[...]
        gt = all_loss > _THRESH
        cnt_i = jnp.sum(gt.astype(jnp.int32))
        sum_gt = jnp.sum(jnp.where(gt, all_loss, 0.0))
        cond = cnt_i >= _TOPN + 1
        mean_thresh = sum_gt / jnp.maximum(cnt_i.astype(jnp.float32), 1.0)

        def mean_topn():
            lin = (
                lax.broadcasted_iota(jnp.int32, all_loss.shape, 0) * _BLOCK_ROWS
                + lax.broadcasted_iota(jnp.int32, all_loss.shape, 1)
            )

            def body(_, carry):
                arr, acc = carry
                mx = jnp.max(arr)
                idx = jnp.min(jnp.where(arr == mx, lin, _N_ROWS))
                arr = jnp.where(lin == idx, -1.0, arr)
                return arr, acc + mx

            _, topsum = lax.fori_loop(0, _TOPN, body, (all_loss, 0.0))
            return topsum / float(_TOPN)

        result = lax.cond(cond, lambda: mean_thresh, mean_topn)
        out_ref[...] = jnp.broadcast_to(result, (1, 1))


@functools.partial(jax.jit, static_argnames=())
def kernel(input, target):
    tgt = target.astype(jnp.int32).reshape(_N_BLOCKS, 1, _BLOCK_ROWS)
    out = pl.pallas_call(
        _ohem_body,
        grid=(_N_BLOCKS,),
        in_specs=[
            pl.BlockSpec((_BLOCK_ROWS, _N_COLS), lambda i: (i, 0)),
            pl.BlockSpec((1, 1, _BLOCK_ROWS), lambda i: (i, 0, 0)),
        ],
        out_specs=pl.BlockSpec((1, 1), lambda i: (0, 0)),
        out_shape=jax.ShapeDtypeStruct((1, 1), jnp.float32),
        scratch_shapes=[pltpu.VMEM((_N_BLOCKS, _BLOCK_ROWS), jnp.float32)],
    )(input, tgt)
    return out[0, 0]


# speedup vs baseline: 2.1014x; 1.1706x over previous
import functools
from math import log

import jax
import jax.numpy as jnp
from jax import lax
from jax.experimental import pallas as pl
from jax.experimental.pallas import tpu as pltpu

_IGNORE_INDEX = -100
_THRESH = -log(0.7)

_N_ROWS = 8192
_N_COLS = 4096
_BLOCK_ROWS = 512
_N_BLOCKS = _N_ROWS // _BLOCK_ROWS
_TOPN = int(_N_ROWS * 0.01)


def _ohem_body(x_ref, tgt_ref, out_ref, loss_ref):
    i = pl.program_id(0)

    x = x_ref[...]
    t = tgt_ref[0, 0, :]

    m = jnp.max(x, axis=1, keepdims=True)
    s = jnp.sum(jnp.exp(x - m), axis=1)
    lse = m[:, 0] + jnp.log(s)

    t_safe = jnp.clip(t, 0, _N_COLS - 1)
    col = lax.broadcasted_iota(jnp.int32, (_BLOCK_ROWS, _N_COLS), 1)
    picked = jnp.sum(jnp.where(col == t_safe[:, None], x, 0.0), axis=1)

    valid = t != _IGNORE_INDEX
    loss = jnp.where(valid, lse - picked, 0.0)
    loss_ref[pl.ds(i, 1), :] = loss[None, :]

    @pl.when(i == _N_BLOCKS - 1)
    def _():
        all_loss = loss_ref[...]
        gt = all_loss > _THRESH
        cnt_i = jnp.sum(gt.astype(jnp.int32))
        sum_gt = jnp.sum(jnp.where(gt, all_loss, 0.0))
        cond = cnt_i >= _TOPN + 1
        mean_thresh = sum_gt / jnp.maximum(cnt_i.astype(jnp.float32), 1.0)

        def mean_topn():
            lin = (
                lax.broadcasted_iota(jnp.int32, all_loss.shape, 0) * _BLOCK_ROWS
                + lax.broadcasted_iota(jnp.int32, all_loss.shape, 1)
            )

            def body(_, carry):
                arr, acc = carry
                mx = jnp.max(arr)
                idx = jnp.min(jnp.where(arr == mx, lin, _N_ROWS))
                arr = jnp.where(lin == idx, -1.0, arr)
                return arr, acc + mx

            _, topsum = lax.fori_loop(0, _TOPN, body, (all_loss, 0.0))
            return topsum / float(_TOPN)

        result = lax.cond(cond, lambda: mean_thresh, mean_topn)
        out_ref[...] = jnp.broadcast_to(result, (1, 1))


@functools.partial(jax.jit, static_argnames=())
def kernel(input, target):
    tgt = target.astype(jnp.int32).reshape(_N_BLOCKS, 1, _BLOCK_ROWS)
    out = pl.pallas_call(
        _ohem_body,
        grid=(_N_BLOCKS,),
        in_specs=[
            pl.BlockSpec((_BLOCK_ROWS, _N_COLS), lambda i: (i, 0)),
            pl.BlockSpec((1, 1, _BLOCK_ROWS), lambda i: (i, 0, 0)),
        ],
        out_specs=pl.BlockSpec((1, 1), lambda i: (0, 0)),
        out_shape=jax.ShapeDtypeStruct((1, 1), jnp.float32),
        scratch_shapes=[pltpu.VMEM((_N_BLOCKS, _BLOCK_ROWS), jnp.float32)],
    )(input, tgt)
    return out[0, 0]


# speedup vs baseline: 2.2239x; 1.0583x over previous
import functools
from math import log

import jax
import jax.numpy as jnp
from jax import lax
from jax.experimental import pallas as pl
from jax.experimental.pallas import tpu as pltpu

_IGNORE_INDEX = -100
_THRESH = -log(0.7)

_N_ROWS = 8192
_N_COLS = 4096
_BLOCK_ROWS = 1024
_N_BLOCKS = _N_ROWS // _BLOCK_ROWS
_TOPN = int(_N_ROWS * 0.01)


def _ohem_body(x_ref, tgt_ref, out_ref, loss_ref):
    i = pl.program_id(0)

    x = x_ref[...]
    t = tgt_ref[0, 0, :]

    m = jnp.max(x, axis=1, keepdims=True)
    s = jnp.sum(jnp.exp(x - m), axis=1)
    lse = m[:, 0] + jnp.log(s)

    t_safe = jnp.clip(t, 0, _N_COLS - 1)
    col = lax.broadcasted_iota(jnp.int32, (_BLOCK_ROWS, _N_COLS), 1)
    picked = jnp.sum(jnp.where(col == t_safe[:, None], x, 0.0), axis=1)

    valid = t != _IGNORE_INDEX
    loss = jnp.where(valid, lse - picked, 0.0)
    loss_ref[pl.ds(i, 1), :] = loss[None, :]

    @pl.when(i == _N_BLOCKS - 1)
    def _():
        all_loss = loss_ref[...]
        gt = all_loss > _THRESH
        cnt_i = jnp.sum(gt.astype(jnp.int32))
        sum_gt = jnp.sum(jnp.where(gt, all_loss, 0.0))
        cond = cnt_i >= _TOPN + 1
        mean_thresh = sum_gt / jnp.maximum(cnt_i.astype(jnp.float32), 1.0)

        def mean_topn():
            lin = (
                lax.broadcasted_iota(jnp.int32, all_loss.shape, 0) * _BLOCK_ROWS
                + lax.broadcasted_iota(jnp.int32, all_loss.shape, 1)
            )

            def body(_, carry):
                arr, acc = carry
                mx = jnp.max(arr)
                idx = jnp.min(jnp.where(arr == mx, lin, _N_ROWS))
                arr = jnp.where(lin == idx, -1.0, arr)
                return arr, acc + mx

            _, topsum = lax.fori_loop(0, _TOPN, body, (all_loss, 0.0))
            return topsum / float(_TOPN)

        result = lax.cond(cond, lambda: mean_thresh, mean_topn)
        out_ref[...] = jnp.broadcast_to(result, (1, 1))


@functools.partial(jax.jit, static_argnames=())
def kernel(input, target):
    tgt = target.astype(jnp.int32).reshape(_N_BLOCKS, 1, _BLOCK_ROWS)
    out = pl.pallas_call(
        _ohem_body,
        grid=(_N_BLOCKS,),
        in_specs=[
            pl.BlockSpec((_BLOCK_ROWS, _N_COLS), lambda i: (i, 0)),
            pl.BlockSpec((1, 1, _BLOCK_ROWS), lambda i: (i, 0, 0)),
        ],
        out_specs=pl.BlockSpec((1, 1), lambda i: (0, 0)),
        out_shape=jax.ShapeDtypeStruct((1, 1), jnp.float32),
        scratch_shapes=[pltpu.VMEM((_N_BLOCKS, _BLOCK_ROWS), jnp.float32)],
    )(input, tgt)
    return out[0, 0]
